# baseline (speedup 1.0000x reference)
import functools

import jax
import jax.numpy as jnp
from jax import lax
from jax.experimental import pallas as pl
from jax.experimental.pallas import tpu as pltpu
from jax.experimental.pallas import tpu_sc as plsc

N1, N2, N3 = 100000, 25000, 6250
C = 128
O = 128

_PACKED = 64


def _pack_bf16(y):
    lo = lax.bitcast_convert_type(
        y[:, :64].astype(jnp.bfloat16), jnp.uint16).astype(jnp.uint32)
    hi = lax.bitcast_convert_type(
        y[:, 64:].astype(jnp.bfloat16), jnp.uint16).astype(jnp.uint32)
    return lax.bitcast_convert_type(lo | (hi << 16), jnp.int32)


def _unpack_bf16(p):
    u = lax.bitcast_convert_type(p, jnp.uint32)
    lo = lax.bitcast_convert_type((u & 0xFFFF).astype(jnp.uint16),
                                  jnp.bfloat16)
    hi = lax.bitcast_convert_type((u >> 16).astype(jnp.uint16),
                                  jnp.bfloat16)
    return jnp.concatenate([lo, hi], axis=1)


def _l3_body(x_ref, w_ref, out_ref):
    y = jnp.dot(x_ref[...], w_ref[...], preferred_element_type=jnp.float32)
    out_ref[...] = jnp.concatenate(
        [_pack_bf16(y[:, :O]), _pack_bf16(y[:, O:])], axis=1)


def _blockdiag(W):
    z = jnp.zeros_like(W)
    return jnp.concatenate([jnp.concatenate([W, z], 1),
                            jnp.concatenate([z, W], 1)], 0)


def _compress_l3(x3, W3, n_out):
    x3p = jnp.concatenate(
        [x3, jnp.zeros((n_out - x3.shape[0], C), jnp.float32)])
    return pl.pallas_call(
        _l3_body,
        in_specs=[pl.BlockSpec((n_out // 2, 2 * C), lambda: (0, 0)),
                  pl.BlockSpec((2 * C, 2 * O), lambda: (0, 0))],
        out_specs=pl.BlockSpec((n_out // 2, 128), lambda: (0, 0)),
        out_shape=jax.ShapeDtypeStruct((n_out // 2, 128), jnp.int32),
    )(x3p.reshape(n_out // 2, 2 * C), _blockdiag(W3))


def _half_tconv(gh, off_ref, wt_ref):
    h = gh.shape[0]
    offc = jnp.transpose(off_ref[0].astype(jnp.bfloat16), (1, 0))
    offb = jnp.broadcast_to(offc, (h, O))
    gm = jnp.concatenate(
        [jnp.where(offb == float(k), gh, 0) for k in range(8)], axis=1)
    return jnp.dot(gm, wt_ref[...], preferred_element_type=jnp.float32)


def _masked_body(pack_out, g_ref, offe_ref, offo_ref, x_ref, w_ref, wt_ref,
                 out_ref):
    u = lax.bitcast_convert_type(g_ref[...], jnp.uint32)
    lo = lax.bitcast_convert_type((u & 0xFFFF).astype(jnp.uint16),
                                  jnp.bfloat16)
    hi = lax.bitcast_convert_type((u >> 16).astype(jnp.uint16),
                                  jnp.bfloat16)
    g_even = jnp.concatenate([lo[:, :64], hi[:, :64]], axis=1)
    g_odd = jnp.concatenate([lo[:, 64:], hi[:, 64:]], axis=1)
    xw = jnp.dot(x_ref[...], w_ref[...],
                 preferred_element_type=jnp.float32)
    y_e = _half_tconv(g_even, offe_ref, wt_ref) + xw[:, :O]
    y_o = _half_tconv(g_odd, offo_ref, wt_ref) + xw[:, O:]
    if pack_out:
        out_ref[...] = jnp.concatenate(
            [_pack_bf16(y_e), _pack_bf16(y_o)], axis=1)
    else:
        out_ref[...] = jnp.concatenate([y_e, y_o], axis=1)


def _masked_level(g, off, x, W, Wt, pack_out, bm, n_out=None):
    n = x.shape[0]
    nm = n // bm
    n_out = n_out or n
    wt_cat = Wt.astype(jnp.bfloat16).reshape(8 * O, O)
    out_shape = (jax.ShapeDtypeStruct((n_out // 2, 128), jnp.int32)
                 if pack_out
                 else jax.ShapeDtypeStruct((n // 2, 2 * O), jnp.float32))
    off2 = off.reshape(nm, 1, bm // 2, 2)
    ow = 128 if pack_out else 2 * O
    return pl.pallas_call(
        functools.partial(_masked_body, pack_out),
        grid=(nm,),
        in_specs=[
            pl.BlockSpec((bm // 2, 128), lambda i: (i, 0)),
            pl.BlockSpec((1, 1, bm // 2), lambda i: (i, 0, 0)),
            pl.BlockSpec((1, 1, bm // 2), lambda i: (i, 0, 0)),
            pl.BlockSpec((bm // 2, 2 * C), lambda i: (i, 0)),
            pl.BlockSpec((2 * C, 2 * O), lambda i: (0, 0)),
            pl.BlockSpec((8 * O, O), lambda i: (0, 0)),
        ],
        out_specs=pl.BlockSpec((bm // 2, ow), lambda i: (i, 0)),
        out_shape=out_shape,
    )(g, off2[..., 0], off2[..., 1],
      x.reshape(n // 2, 2 * C), _blockdiag(W), wt_cat)


def _sc_gather(table, parent, br, nbuf, depth):
    info = plsc.get_sparse_core_info()
    nw = info.num_cores * info.num_subcores
    ns = info.num_subcores
    n = parent.shape[0]
    pw = br * -(-n // (br * nw))
    n_pad = pw * nw
    nk = pw // br

    vp = table.shape[0]
    assert vp % 128 == 0
    vs = vp // ns
    parent_p = jnp.concatenate([parent, jnp.zeros((n_pad - n,), jnp.int32)])

    mesh = plsc.VectorSubcoreMesh(core_axis_name="c", subcore_axis_name="s")

    @functools.partial(
        pl.kernel,
        out_type=jax.ShapeDtypeStruct((n_pad, _PACKED), jnp.int32),
        mesh=mesh,
        compiler_params=pltpu.CompilerParams(use_tc_tiling_on_sc=False),
        scratch_types=[
            pltpu.VMEM_SHARED((vp, _PACKED), jnp.int32),
            pltpu.VMEM((pw,), jnp.int32),
            pltpu.VMEM((nbuf, br, _PACKED), jnp.int32),
            pltpu.SemaphoreType.DMA,
            pltpu.SemaphoreType.DMA,
        ],
    )
    def gather(table_hbm, par_hbm, out_hbm,
               spmem, idx_v, bufs, sem_g, sem_s):
        sid = lax.axis_index("s")
        wid = sid * info.num_cores + lax.axis_index("c")
        base = wid * pw
        pltpu.sync_copy(table_hbm.at[pl.ds(sid * vs, vs)],
                        spmem.at[pl.ds(sid * vs, vs)])
        pltpu.sync_copy(par_hbm.at[pl.ds(base, pw)], idx_v)
        plsc.subcore_barrier()
        g_copies = [None] * nk
        s_copies = [None] * nk
        dp = min(depth, nk)
        for j in range(nk + dp):
            if j < nk:
                if j >= nbuf:
                    s_copies[j - nbuf].wait()
                g_copies[j] = pltpu.async_copy(
                    spmem.at[idx_v.at[pl.ds(j * br, br)]],
                    bufs.at[j % nbuf], sem_g)
            t = j - dp
            if t >= 0:
                g_copies[t].wait()
                s_copies[t] = pltpu.async_copy(
                    bufs.at[t % nbuf],
                    out_hbm.at[pl.ds(base + t * br, br)],
                    sem_s)
        for t in range(max(0, nk - nbuf), nk):
            s_copies[t].wait()

    return gather(table, parent_p)


def kernel(x1, x2, x3, parent1, offset1, parent2, offset2,
           W1, W2, W3, Wt2, Wt3):
    parent1 = parent1.astype(jnp.int32)
    offset1 = offset1.astype(jnp.int32)
    parent2 = parent2.astype(jnp.int32)
    offset2 = offset2.astype(jnp.int32)

    vp3 = 128 * -(-N3 // 128)
    vp2 = 128 * -(-N2 // 128)
    y3p = _compress_l3(x3, W3, vp3)
    g3 = _sc_gather(y3p.reshape(vp3, _PACKED),
                    parent2, 128, 6, 4)
    g3 = g3.reshape(g3.shape[0] // 2, 128)
    x2p = jnp.concatenate(
        [x2, jnp.zeros((vp2 - N2, C), jnp.float32)])
    off2p = jnp.concatenate(
        [offset2, jnp.zeros((vp2 - N2,), jnp.int32)])
    y2p = _masked_level(g3, off2p, x2p, W2, Wt3,
                        pack_out=True, bm=6272, n_out=vp2)
    g2 = _sc_gather(y2p.reshape(vp2, _PACKED),
                    parent1, 128, 2, 1)
    g2 = g2.reshape(g2.shape[0] // 2, 128)
    out = _masked_level(g2, offset1, x1, W1, Wt2,
                         pack_out=False, bm=10000)
    return out.reshape(N1, O)


# speedup vs baseline: 1.5089x; 1.5089x over previous
import functools

import jax
import jax.numpy as jnp
from jax import lax
from jax.experimental import pallas as pl
from jax.experimental.pallas import tpu as pltpu
from jax.experimental.pallas import tpu_sc as plsc

N1, N2, N3 = 100000, 25000, 6250
C = 128
O = 128

_PACKED = 64


def _pack_bf16(y):
    lo = lax.bitcast_convert_type(
        y[:, :64].astype(jnp.bfloat16), jnp.uint16).astype(jnp.uint32)
    hi = lax.bitcast_convert_type(
        y[:, 64:].astype(jnp.bfloat16), jnp.uint16).astype(jnp.uint32)
    return lax.bitcast_convert_type(lo | (hi << 16), jnp.int32)


def _unpack_bf16(p):
    u = lax.bitcast_convert_type(p, jnp.uint32)
    lo = lax.bitcast_convert_type((u & 0xFFFF).astype(jnp.uint16),
                                  jnp.bfloat16)
    hi = lax.bitcast_convert_type((u >> 16).astype(jnp.uint16),
                                  jnp.bfloat16)
    return jnp.concatenate([lo, hi], axis=1)


def _l3_body(x_ref, w_ref, out_ref):
    out_ref[...] = _pack_bf16(
        jnp.dot(x_ref[...], w_ref[...], preferred_element_type=jnp.float32))


def _compress_l3(x3, W3, n_out):
    x3p = jnp.concatenate(
        [x3, jnp.zeros((n_out - x3.shape[0], C), jnp.float32)])
    return pl.pallas_call(
        _l3_body,
        in_specs=[pl.BlockSpec((n_out, C), lambda: (0, 0)),
                  pl.BlockSpec((C, O), lambda: (0, 0))],
        out_specs=pl.BlockSpec((n_out, _PACKED), lambda: (0, 0)),
        out_shape=jax.ShapeDtypeStruct((n_out, _PACKED), jnp.int32),
    )(x3p, W3)


def _masked_body(pack_out, g_ref, off_ref, x_ref, w_ref, wt_ref, out_ref):
    g = _unpack_bf16(g_ref[...])
    bm = g.shape[0]
    offc = jnp.transpose(off_ref[0].astype(jnp.bfloat16), (1, 0))
    offb = jnp.broadcast_to(offc, (bm, O))
    gm = jnp.concatenate(
        [jnp.where(offb == float(k), g, 0) for k in range(8)],
        axis=1)
    y = jnp.dot(gm, wt_ref[...], preferred_element_type=jnp.float32)
    y = y + jnp.dot(x_ref[...], w_ref[...],
                    preferred_element_type=jnp.float32)
    out_ref[...] = _pack_bf16(y) if pack_out else y


def _masked_level(g, off, x, W, Wt, pack_out, bm, n_out=None):
    n = x.shape[0]
    nm = n // bm
    n_out = n_out or n
    wt_cat = Wt.astype(jnp.bfloat16).reshape(8 * O, O)
    out_shape = (jax.ShapeDtypeStruct((n_out, _PACKED), jnp.int32)
                 if pack_out else jax.ShapeDtypeStruct((n, O), jnp.float32))
    ow = _PACKED if pack_out else O
    return pl.pallas_call(
        functools.partial(_masked_body, pack_out),
        grid=(nm,),
        in_specs=[
            pl.BlockSpec((bm, _PACKED), lambda i: (i, 0)),
            pl.BlockSpec((1, 1, bm), lambda i: (i, 0, 0)),
            pl.BlockSpec((bm, C), lambda i: (i, 0)),
            pl.BlockSpec((C, O), lambda i: (0, 0)),
            pl.BlockSpec((8 * O, O), lambda i: (0, 0)),
        ],
        out_specs=pl.BlockSpec((bm, ow), lambda i: (i, 0)),
        out_shape=out_shape,
    )(g, off.reshape(nm, 1, bm), x, W, wt_cat)


def _sc_gather(table, parent, br, nbuf, depth):
    info = plsc.get_sparse_core_info()
    nw = info.num_cores * info.num_subcores
    ns = info.num_subcores
    n = parent.shape[0]
    pw = br * -(-n // (br * nw))
    n_pad = pw * nw
    nk = pw // br

    vp = table.shape[0]
    assert vp % 128 == 0
    vs = vp // ns
    parent_p = jnp.concatenate([parent, jnp.zeros((n_pad - n,), jnp.int32)])

    mesh = plsc.VectorSubcoreMesh(core_axis_name="c", subcore_axis_name="s")

    @functools.partial(
        pl.kernel,
        out_type=jax.ShapeDtypeStruct((n_pad, _PACKED), jnp.int32),
        mesh=mesh,
        compiler_params=pltpu.CompilerParams(use_tc_tiling_on_sc=False),
        scratch_types=[
            pltpu.VMEM_SHARED((vp, _PACKED), jnp.int32),
            pltpu.VMEM((pw,), jnp.int32),
            pltpu.VMEM((nbuf, br, _PACKED), jnp.int32),
            pltpu.SemaphoreType.DMA,
            pltpu.SemaphoreType.DMA,
        ],
    )
    def gather(table_hbm, par_hbm, out_hbm,
               spmem, idx_v, bufs, sem_g, sem_s):
        sid = lax.axis_index("s")
        wid = sid * info.num_cores + lax.axis_index("c")
        base = wid * pw
        pltpu.sync_copy(table_hbm.at[pl.ds(sid * vs, vs)],
                        spmem.at[pl.ds(sid * vs, vs)])
        pltpu.sync_copy(par_hbm.at[pl.ds(base, pw)], idx_v)
        plsc.subcore_barrier()
        g_copies = [None] * nk
        s_copies = [None] * nk
        dp = min(depth, nk)
        for j in range(nk + dp):
            if j < nk:
                if j >= nbuf:
                    s_copies[j - nbuf].wait()
                g_copies[j] = pltpu.async_copy(
                    spmem.at[idx_v.at[pl.ds(j * br, br)]],
                    bufs.at[j % nbuf], sem_g)
            t = j - dp
            if t >= 0:
                g_copies[t].wait()
                s_copies[t] = pltpu.async_copy(
                    bufs.at[t % nbuf],
                    out_hbm.at[pl.ds(base + t * br, br)],
                    sem_s)
        for t in range(max(0, nk - nbuf), nk):
            s_copies[t].wait()

    return gather(table, parent_p)


def kernel(x1, x2, x3, parent1, offset1, parent2, offset2,
           W1, W2, W3, Wt2, Wt3):
    parent1 = parent1.astype(jnp.int32)
    offset1 = offset1.astype(jnp.int32)
    parent2 = parent2.astype(jnp.int32)
    offset2 = offset2.astype(jnp.int32)

    vp3 = 128 * -(-N3 // 128)
    vp2 = 128 * -(-N2 // 128)
    y3p = _compress_l3(x3, W3, vp3)
    g3 = _sc_gather(y3p, parent2, 128, 6, 4)
    y2p = _masked_level(g3, offset2, x2, W2, Wt3,
                        pack_out=True, bm=5000, n_out=vp2)
    g2 = _sc_gather(y2p, parent1, 128, 2, 1)
    return _masked_level(g2, offset1, x1, W1, Wt2,
                         pack_out=False, bm=5000)
